# Initial kernel scaffold; baseline (speedup 1.0000x reference)
import jax
import jax.numpy as jnp
from jax.experimental import pallas as pl


def kernel(feat_item, feat_category, feat_brand, positions, table_item, table_category, table_brand, table_position):
    raise NotImplementedError("write your pallas kernel here")


# speedup vs baseline: 6.5512x; 6.5512x over previous
import functools

import jax
import jax.numpy as jnp
from jax import lax
from jax.experimental import pallas as pl
from jax.experimental.pallas import tpu as pltpu
from jax.experimental.pallas import tpu_sc as plsc

DIM = 64
CHUNK = 400


def _make_sc_kernel(n_rows: int, n_workers: int):
    rows_per_w = n_rows // n_workers
    n_chunks = rows_per_w // CHUNK
    mesh = plsc.VectorSubcoreMesh(core_axis_name="c", subcore_axis_name="s")

    @functools.partial(
        pl.kernel,
        mesh=mesh,
        compiler_params=pltpu.CompilerParams(use_tc_tiling_on_sc=False),
        out_type=jax.ShapeDtypeStruct((n_rows, DIM), jnp.float32),
        scratch_types=[
            pltpu.VMEM((CHUNK,), jnp.int32),
            pltpu.VMEM((CHUNK,), jnp.int32),
            pltpu.VMEM((CHUNK,), jnp.int32),
            pltpu.VMEM((CHUNK, DIM), jnp.float32),
            pltpu.VMEM((CHUNK, DIM), jnp.float32),
            pltpu.VMEM((CHUNK, DIM), jnp.float32),
            pltpu.VMEM((CHUNK, DIM), jnp.float32),
            pltpu.SemaphoreType.DMA,
            pltpu.SemaphoreType.DMA,
            pltpu.SemaphoreType.DMA,
        ],
    )
    def sc_kernel(idx_a_hbm, idx_b_hbm, idx_c_hbm, tab_a_hbm, tab_b_hbm,
                  tab_c_hbm, pos_hbm, out_hbm,
                  idx_a, idx_b, idx_c, rows_a, rows_b, rows_c, pos_v,
                  sem_a, sem_b, sem_c):
        n_cores = 2
        wid = lax.axis_index("s") * n_cores + lax.axis_index("c")
        w_base = wid * rows_per_w

        pltpu.sync_copy(pos_hbm, pos_v)

        def do_chunk(ci, carry):
            base = w_base + ci * CHUNK
            sl = pl.ds(base, CHUNK)
            pltpu.sync_copy(idx_a_hbm.at[sl], idx_a)
            pltpu.sync_copy(idx_b_hbm.at[sl], idx_b)
            pltpu.sync_copy(idx_c_hbm.at[sl], idx_c)
            cp_a = pltpu.async_copy(tab_a_hbm.at[idx_a], rows_a, sem_a)
            cp_b = pltpu.async_copy(tab_b_hbm.at[idx_b], rows_b, sem_b)
            cp_c = pltpu.async_copy(tab_c_hbm.at[idx_c], rows_c, sem_c)
            cp_a.wait()
            cp_b.wait()
            cp_c.wait()

            def row_body(r, carry2):
                for k in range(DIM // 16):
                    s = pl.ds(k * 16, 16)
                    rows_a[r, s] = (rows_a[r, s] + rows_b[r, s]
                                    + rows_c[r, s] + pos_v[r, s])
                return carry2

            lax.fori_loop(0, CHUNK, row_body, 0)
            pltpu.sync_copy(rows_a, out_hbm.at[sl])
            return carry

        lax.fori_loop(0, n_chunks, do_chunk, 0)

    return sc_kernel


def kernel(feat_item, feat_category, feat_brand, positions,
           table_item, table_category, table_brand, table_position):
    batch, pos_len = feat_item.shape
    n_rows = batch * pos_len

    idx_a = feat_item.reshape(n_rows)
    idx_b = feat_category.reshape(n_rows)
    idx_c = feat_brand.reshape(n_rows)

    pos_rows = jnp.take(table_position, positions, axis=0)
    pos_block = jnp.tile(pos_rows, (CHUNK // pos_len, 1))

    sc = _make_sc_kernel(n_rows, 32)
    out = sc(idx_a, idx_b, idx_c, table_item, table_category,
             table_brand, pos_block)
    return out.reshape(batch, pos_len, DIM)
